# baseline (speedup 1.0000x reference)
import functools

import jax
import jax.numpy as jnp
from jax import lax
from jax.experimental import pallas as pl
from jax.experimental.pallas import tpu as pltpu
from jax.experimental.pallas import tpu_sc as plsc

N = 10000
E = 160000
P = 3
D = 256
HF = 128
NC = 2
NS = 16
GPT = 80
EPAD = NS * GPT * 128
NPADE = EPAD - E
ACC_ROWS = 10240
AH = 5248
HNR = 5120
EGRP = 2048
R = 2000


_SC_CACHE = {}


def _sc_kernels():
    if "deg" in _SC_CACHE:
        return _SC_CACHE["deg"], _SC_CACHE["agg"]

    mesh = plsc.VectorSubcoreMesh(
        core_axis_name="c", subcore_axis_name="s", num_cores=NC, num_subcores=NS
    )

    @functools.partial(
        pl.kernel,
        mesh=mesh,
        out_type=jax.ShapeDtypeStruct((6, ACC_ROWS), jnp.float32),
        scratch_types=[
            pltpu.VMEM((GPT, 128), jnp.int32),
            pltpu.VMEM((128,), jnp.float32),
            pltpu.VMEM((ACC_ROWS // NS,), jnp.float32),
            pltpu.VMEM_SHARED((ACC_ROWS,), jnp.float32),
            pltpu.VMEM_SHARED((ACC_ROWS,), jnp.float32),
            pltpu.VMEM_SHARED((ACC_ROWS,), jnp.float32),
        ],
    )
    def _deg_kernel(eir, deg_out, ibuf, ones_v, zer_v, d0, d1, d2):
        c = lax.axis_index("c")
        s = lax.axis_index("s")
        dlist = (d0, d1, d2)
        pltpu.sync_copy(eir.at[0, pl.ds(EGRP - 8, 8)], ibuf.at[pl.ds(0, 8)])
        z = pl.multiple_of(ibuf[0, pl.ds(0, 16)][0], 8)
        for k in range(128 // 16):
            ones_v[pl.ds(k * 16, 16)] = jnp.ones((16,), jnp.float32)
        for k in range((ACC_ROWS // NS) // 16):
            zer_v[pl.ds(k * 16, 16)] = jnp.zeros((16,), jnp.float32)
        for j in range(P):
            pltpu.sync_copy(
                zer_v, dlist[j].at[pl.ds(s * (ACC_ROWS // NS), ACC_ROWS // NS)]
            )
        plsc.subcore_barrier()
        for j in range(P):
            pltpu.sync_copy(eir.at[c * 3 + j, pl.ds(s * GPT + z, GPT)], ibuf)

            def body(g, carry, _j=j):
                pltpu.sync_copy(ones_v, dlist[_j].at[ibuf.at[g]], add=True)
                return carry

            lax.fori_loop(0, GPT, body, 0)
        plsc.subcore_barrier()
        for j in range(P):
            @pl.when(s == j)
            def _(_j=j):
                pltpu.sync_copy(dlist[_j], deg_out.at[c * 3 + _j])

    @functools.partial(
        pl.kernel,
        mesh=mesh,
        out_type=jax.ShapeDtypeStruct((6, N, HF), jnp.float32),
        scratch_types=[
            pltpu.VMEM((GPT, 128), jnp.int32),
            pltpu.VMEM((GPT, 128), jnp.int32),
            pltpu.VMEM((4, 128), jnp.int32),
            pltpu.VMEM((4, 128), jnp.int32),
            pltpu.VMEM((128, HF), jnp.float32),
            pltpu.VMEM((128, HF), jnp.float32),
            pltpu.VMEM((64, HF), jnp.float32),
            pltpu.VMEM_SHARED((AH, HF), jnp.float32),
            pltpu.SemaphoreType.DMA,
            pltpu.SemaphoreType.DMA,
            pltpu.SemaphoreType.DMA,
            pltpu.SemaphoreType.DMA,
        ],
    )
    def _agg_kernel(eir, tabh, aggh3, sbuf, dbuf, gt, dt,
                    r0, r1, zbuf, acc, gs0, gs1, ss0, ss1):
        c = lax.axis_index("c")
        s = lax.axis_index("s")

        pltpu.sync_copy(eir.at[0, pl.ds(EGRP - 8, 8)], sbuf.at[pl.ds(0, 8)])
        z = pl.multiple_of(sbuf[0, pl.ds(0, 16)][0], 8)

        def zrow(rr, carry):
            for k in range(HF // 16):
                zbuf[rr, pl.ds(k * 16, 16)] = jnp.zeros((16,), jnp.float32)
            return carry

        lax.fori_loop(0, 64, zrow, 0)

        for i in range(P):
            pltpu.sync_copy(eir.at[i, pl.ds(s * GPT + z, GPT)], sbuf)
            pltpu.sync_copy(eir.at[3 + i, pl.ds(s * GPT + z, GPT)], dbuf)
            jrow = 2 * i + c
            base = jrow * N
            for p in range(2):
                lo = p * HNR
                for k in range(5):
                    pltpu.sync_copy(zbuf, acc.at[pl.ds(s * 328 + k * 64, 64)])
                pltpu.sync_copy(
                    zbuf.at[pl.ds(0, 8)], acc.at[pl.ds(s * 328 + 320, 8)]
                )
                plsc.subcore_barrier()

                def gidx(b):
                    return plsc.Indices(gt.at[b], ignored_value=-1)

                def didx(b):
                    return plsc.Indices(dt.at[b], ignored_value=-1)

                rbufs = (r0, r1)
                gsems = (gs0, gs1)
                ssems = (ss0, ss1)

                def chunk(ck, carry, lo=lo, base=base):
                    g0 = 2 * ck
                    for b in range(2):
                        for k in range(8):
                            sv = sbuf[g0 + b, pl.ds(k * 16, 16)]
                            dv = dbuf[g0 + b, pl.ds(k * 16, 16)] - lo
                            inr = (dv >= 0) & (dv < HNR)
                            gt[b, pl.ds(k * 16, 16)] = jnp.where(
                                inr, sv + base, -1)
                            dt[b, pl.ds(k * 16, 16)] = jnp.where(inr, dv, -1)
                        pltpu.async_copy(
                            tabh.at[gidx(b)], rbufs[b], gsems[b])
                    for b in range(2):
                        pltpu.make_async_copy(
                            tabh.at[gidx(b)], rbufs[b], gsems[b]).wait()
                        pltpu.async_copy(
                            rbufs[b], acc.at[didx(b)], ssems[b], add=True)
                    for b in range(2):
                        pltpu.make_async_copy(
                            rbufs[b], acc.at[didx(b)], ssems[b]).wait()
                    return carry

                lax.fori_loop(0, GPT // 2, chunk, 0)
                plsc.subcore_barrier()
                cnt = 512 if p == 0 else 488
                @pl.when(s < 10)
                def _(jrow=jrow, lo=lo, cnt=cnt):
                    pltpu.sync_copy(
                        acc.at[pl.ds(s * cnt, cnt)],
                        aggh3.at[jrow, pl.ds(lo + s * cnt, cnt)],
                    )
                plsc.subcore_barrier()

    _SC_CACHE["deg"] = _deg_kernel
    _SC_CACHE["agg"] = _agg_kernel
    return _SC_CACHE["deg"], _SC_CACHE["agg"]


def _mm(a, b):
    return lax.dot_general(
        a, b, (((1,), (0,)), ((), ())),
        preferred_element_type=jnp.float32,
        precision=lax.Precision.HIGHEST,
    )


def _tab_halves(x, dpack, w_ref, out_ref, out_dt):
    for i in range(P):
        deg = dpack[:, i] - dpack[:, 6]
        nsrc = lax.rsqrt(jnp.maximum(deg, 1.0))
        hw = _mm(x * nsrc[:, None], w_ref[i]).astype(out_dt)
        out_ref[2 * i] = hw[:, :HF]
        out_ref[2 * i + 1] = hw[:, HF:]


def _a0_body(x_ref, dpack_ref, w_ref, out_ref):
    _tab_halves(x_ref[...], dpack_ref[...], w_ref, out_ref, jnp.float32)


def _combine(emb_ref, wacc_ref, sq_ref):
    wv = jnp.sum(wacc_ref[...] * sq_ref[...], axis=1, keepdims=True) / N
    m = jnp.max(wv)
    ex = jnp.exp(wv - m)
    beta = ex / jnp.sum(ex)
    return (
        emb_ref[0] * beta[0:1]
        + emb_ref[1] * beta[1:2]
        + emb_ref[2] * beta[2:3]
    )


def _a1_body(emb_ref, wacc_ref, sq_ref, dpack_ref, w_ref, out_ref):
    x = _combine(emb_ref, wacc_ref, sq_ref)
    _tab_halves(x, dpack_ref[...], w_ref, out_ref, jnp.float32)


def _c_body(agg_ref, dpack_ref, b_ref, sw_ref, sb_ref, emb_ref, wacc_ref):
    @pl.when(pl.program_id(0) == 0)
    def _():
        wacc_ref[...] = jnp.zeros_like(wacc_ref)

    for i in range(P):
        a = jnp.concatenate(
            [agg_ref[2 * i], agg_ref[2 * i + 1]], axis=1
        ).astype(jnp.float32)
        ndst = lax.rsqrt(jnp.maximum(dpack_ref[:, 3 + i], 1.0))
        e = a * ndst[:, None] + b_ref[i, :][None, :]
        e = jnp.where(e > 0, e, jnp.exp(jnp.minimum(e, 0.0)) - 1.0)
        emb_ref[i] = e
        t = jnp.tanh(_mm(e, sw_ref[...]) + sb_ref[...])
        wacc_ref[i, :] += jnp.sum(t, axis=0)


def _d1_body(emb_ref, wacc_ref, sq_ref, num_ref, w1_ref, b1_ref, w2_ref,
             b2_ref, w3_ref, b3_ref, out_ref):
    x = _combine(emb_ref, wacc_ref, sq_ref) * num_ref[...]
    h1 = jnp.maximum(_mm(x, w1_ref[...]) + b1_ref[...], 0.0)
    h2 = jnp.maximum(_mm(h1, w2_ref[...]) + b2_ref[...], 0.0)
    out_ref[...] = _mm(h2, w3_ref[...]) + b3_ref[...]


_GRID = (N // R,)
_FULL = lambda *shape: pl.BlockSpec(shape, lambda r: tuple(0 for _ in shape))
_DPACK_SPEC = lambda: pl.BlockSpec((R, 8), lambda r: (r, 0))


def _tc_a0(x, dpack, w):
    return pl.pallas_call(
        _a0_body,
        grid=_GRID,
        in_specs=[
            pl.BlockSpec((R, D), lambda r: (r, 0)),
            _DPACK_SPEC(),
            _FULL(P, D, D),
        ],
        out_specs=pl.BlockSpec((6, R, HF), lambda r: (0, r, 0)),
        out_shape=jax.ShapeDtypeStruct((6, N, HF), jnp.float32),
    )(x, dpack, w)


def _tc_a1(emb, wacc, sq, dpack, w):
    return pl.pallas_call(
        _a1_body,
        grid=_GRID,
        in_specs=[
            pl.BlockSpec((P, R, D), lambda r: (0, r, 0)),
            _FULL(P, D),
            _FULL(1, D),
            _DPACK_SPEC(),
            _FULL(P, D, D),
        ],
        out_specs=pl.BlockSpec((6, R, HF), lambda r: (0, r, 0)),
        out_shape=jax.ShapeDtypeStruct((6, N, HF), jnp.float32),
    )(emb, wacc, sq, dpack, w)


def _tc_c(agg, dpack, b, sw, sb):
    return pl.pallas_call(
        _c_body,
        grid=_GRID,
        in_specs=[
            pl.BlockSpec((6, R, HF), lambda r: (0, r, 0)),
            _DPACK_SPEC(),
            _FULL(P, D),
            _FULL(D, D),
            _FULL(1, D),
        ],
        out_specs=[
            pl.BlockSpec((P, R, D), lambda r: (0, r, 0)),
            _FULL(P, D),
        ],
        out_shape=[
            jax.ShapeDtypeStruct((P, N, D), jnp.float32),
            jax.ShapeDtypeStruct((P, D), jnp.float32),
        ],
    )(agg, dpack, b, sw, sb)


def _tc_d1(emb, wacc, sq, numrow, w1, b1, w2, b2, w3, b3):
    return pl.pallas_call(
        _d1_body,
        grid=_GRID,
        in_specs=[
            pl.BlockSpec((P, R, D), lambda r: (0, r, 0)),
            _FULL(P, D),
            _FULL(1, D),
            _FULL(1, D),
            _FULL(D, D),
            _FULL(1, D),
            _FULL(D, 128),
            _FULL(1, 128),
            _FULL(128, 64),
            _FULL(1, 64),
        ],
        out_specs=pl.BlockSpec((R, 64), lambda r: (r, 0)),
        out_shape=jax.ShapeDtypeStruct((N, 64), jnp.float32),
    )(emb, wacc, sq, numrow, w1, b1, w2, b2, w3, b3)


def kernel(h, edge_index_0, edge_index_1, edge_index_2,
           gcn_w_l0, gcn_b_l0, gcn_w_l1, gcn_b_l1,
           sa_w_l0, sa_b_l0, sa_q_l0, sa_w_l1, sa_b_l1, sa_q_l1,
           num, pw1, pb1, pw2, pb2, pw3, pb3):
    pad_src = (jnp.arange(NPADE, dtype=jnp.int32) % 64)
    pad_dst = N + (jnp.arange(NPADE, dtype=jnp.int32) % 64)
    eis = (edge_index_0, edge_index_1, edge_index_2)
    rows = [jnp.concatenate([ei[0], pad_src]) for ei in eis]
    rows += [jnp.concatenate([ei[1], pad_dst]) for ei in eis]
    eir = jnp.stack(rows).reshape(6, EPAD // 128, 128)
    eir = jnp.pad(eir, ((0, 0), (0, EGRP - EPAD // 128), (0, 0)))

    fix = jnp.zeros((N, 1), jnp.float32).at[:64, 0].set(NPADE // 64)

    scale = jnp.arange(D) // (D // 10)
    numrow = num[:, scale]

    sb0 = sa_b_l0.reshape(1, D)
    sb1 = sa_b_l1.reshape(1, D)
    sq0 = sa_q_l0.reshape(1, D)
    sq1 = sa_q_l1.reshape(1, D)
    w1p = jnp.pad(pw1, ((0, 0), (0, 6)))
    b1p = jnp.pad(pb1, (0, 6)).reshape(1, D)
    w2p = jnp.pad(pw2, ((0, 6), (0, 53)))
    b2p = jnp.pad(pb2, (0, 53)).reshape(1, 128)
    w3p = jnp.pad(pw3, ((0, 53), (0, 0)))
    b3p = pb3.reshape(1, 64)

    deg_kernel, agg_kernel = _sc_kernels()
    degs = deg_kernel(eir)[:, :N]
    dpack = jnp.concatenate([degs.T, fix, jnp.zeros((N, 1), jnp.float32)], axis=1)

    tab0 = _tc_a0(h, dpack, gcn_w_l0)
    agg0 = agg_kernel(eir, tab0.reshape(6 * N, HF))
    emb0, wacc0 = _tc_c(agg0, dpack, gcn_b_l0, sa_w_l0, sb0)

    tab1 = _tc_a1(emb0, wacc0, sq0, dpack, gcn_w_l1)
    agg1 = agg_kernel(eir, tab1.reshape(6 * N, HF))
    emb1, wacc1 = _tc_c(agg1, dpack, gcn_b_l1, sa_w_l1, sb1)

    return _tc_d1(emb1, wacc1, sq1, numrow, w1p, b1p, w2p, b2p, w3p, b3p)


# speedup vs baseline: 1.3534x; 1.3534x over previous
import functools

import jax
import jax.numpy as jnp
from jax import lax
from jax.experimental import pallas as pl
from jax.experimental.pallas import tpu as pltpu
from jax.experimental.pallas import tpu_sc as plsc

N = 10000
E = 160000
P = 3
D = 256
HF = 128
NC = 2
NS = 16
GPT = 80
EPAD = NS * GPT * 128
NPADE = EPAD - E
ACC_ROWS = 10240
AH = 5248
HNR = 5120
EGRP = 2048
R = 2000


_SC_CACHE = {}


def _sc_kernels():
    if "deg" in _SC_CACHE:
        return _SC_CACHE["deg"], _SC_CACHE["agg"]

    mesh = plsc.VectorSubcoreMesh(
        core_axis_name="c", subcore_axis_name="s", num_cores=NC, num_subcores=NS
    )

    @functools.partial(
        pl.kernel,
        mesh=mesh,
        out_type=jax.ShapeDtypeStruct((6, ACC_ROWS), jnp.float32),
        scratch_types=[
            pltpu.VMEM((GPT, 128), jnp.int32),
            pltpu.VMEM((128,), jnp.float32),
            pltpu.VMEM((ACC_ROWS // NS,), jnp.float32),
            pltpu.VMEM_SHARED((ACC_ROWS,), jnp.float32),
            pltpu.VMEM_SHARED((ACC_ROWS,), jnp.float32),
            pltpu.VMEM_SHARED((ACC_ROWS,), jnp.float32),
        ],
    )
    def _deg_kernel(eir, deg_out, ibuf, ones_v, zer_v, d0, d1, d2):
        c = lax.axis_index("c")
        s = lax.axis_index("s")
        dlist = (d0, d1, d2)
        pltpu.sync_copy(eir.at[0, pl.ds(EGRP - 8, 8)], ibuf.at[pl.ds(0, 8)])
        z = pl.multiple_of(ibuf[0, pl.ds(0, 16)][0], 8)
        for k in range(128 // 16):
            ones_v[pl.ds(k * 16, 16)] = jnp.ones((16,), jnp.float32)
        for k in range((ACC_ROWS // NS) // 16):
            zer_v[pl.ds(k * 16, 16)] = jnp.zeros((16,), jnp.float32)
        for j in range(P):
            pltpu.sync_copy(
                zer_v, dlist[j].at[pl.ds(s * (ACC_ROWS // NS), ACC_ROWS // NS)]
            )
        plsc.subcore_barrier()
        for j in range(P):
            pltpu.sync_copy(eir.at[c * 3 + j, pl.ds(s * GPT + z, GPT)], ibuf)

            def body(g, carry, _j=j):
                pltpu.sync_copy(ones_v, dlist[_j].at[ibuf.at[g]], add=True)
                return carry

            lax.fori_loop(0, GPT, body, 0)
        plsc.subcore_barrier()
        for j in range(P):
            @pl.when(s == j)
            def _(_j=j):
                pltpu.sync_copy(dlist[_j], deg_out.at[c * 3 + _j])

    @functools.partial(
        pl.kernel,
        mesh=mesh,
        out_type=jax.ShapeDtypeStruct((6, N, HF), jnp.float32),
        scratch_types=[
            pltpu.VMEM((GPT, 128), jnp.int32),
            pltpu.VMEM((GPT, 128), jnp.int32),
            pltpu.VMEM((GPT, 128), jnp.int32),
            pltpu.VMEM((GPT, 128), jnp.int32),
            pltpu.VMEM((128, HF), jnp.float32),
            pltpu.VMEM((128, HF), jnp.float32),
            pltpu.VMEM((64, HF), jnp.float32),
            pltpu.VMEM_SHARED((AH, HF), jnp.float32),
            pltpu.SemaphoreType.DMA,
            pltpu.SemaphoreType.DMA,
        ],
    )
    def _agg_kernel(eir, tabh, aggh3, sbuf, dbuf, gbuf2, dbuf2, rA, rB,
                    zbuf, acc, semA, semB):
        c = lax.axis_index("c")
        s = lax.axis_index("s")

        pltpu.sync_copy(eir.at[0, pl.ds(EGRP - 8, 8)], sbuf.at[pl.ds(0, 8)])
        z = pl.multiple_of(sbuf[0, pl.ds(0, 16)][0], 8)

        def zrow(rr, carry):
            for k in range(HF // 16):
                zbuf[rr, pl.ds(k * 16, 16)] = jnp.zeros((16,), jnp.float32)
            return carry

        lax.fori_loop(0, 64, zrow, 0)

        for i in range(P):
            pltpu.sync_copy(eir.at[i, pl.ds(s * GPT + z, GPT)], sbuf)
            pltpu.sync_copy(eir.at[3 + i, pl.ds(s * GPT + z, GPT)], dbuf)
            jrow = 2 * i + c
            base = jrow * N
            for p in range(2):
                lo = p * HNR
                def remap(gg, carry, lo=lo, base=base):
                    for k in range(8):
                        sv = sbuf[gg, pl.ds(k * 16, 16)]
                        dv = dbuf[gg, pl.ds(k * 16, 16)] - lo
                        inr = (dv >= 0) & (dv < HNR)
                        gbuf2[gg, pl.ds(k * 16, 16)] = jnp.where(
                            inr, sv + base, -1)
                        dbuf2[gg, pl.ds(k * 16, 16)] = jnp.where(inr, dv, -1)
                    return carry

                lax.fori_loop(0, GPT, remap, 0)
                for k in range(5):
                    pltpu.sync_copy(zbuf, acc.at[pl.ds(s * 328 + k * 64, 64)])
                pltpu.sync_copy(
                    zbuf.at[pl.ds(0, 8)], acc.at[pl.ds(s * 328 + 320, 8)]
                )
                plsc.subcore_barrier()

                def gidx(g):
                    return plsc.Indices(gbuf2.at[g], ignored_value=-1)

                def didx(g):
                    return plsc.Indices(dbuf2.at[g], ignored_value=-1)

                pltpu.async_copy(tabh.at[gidx(0)], rA, semA)

                def body(g2, carry):
                    g = 2 * g2
                    pltpu.async_copy(tabh.at[gidx(g + 1)], rB, semB)
                    pltpu.make_async_copy(tabh.at[gidx(g)], rA, semA).wait()
                    pltpu.sync_copy(rA, acc.at[didx(g)], add=True)
                    @pl.when(g2 < GPT // 2 - 1)
                    def _():
                        pltpu.async_copy(tabh.at[gidx(g + 2)], rA, semA)
                    pltpu.make_async_copy(tabh.at[gidx(g + 1)], rB, semB).wait()
                    pltpu.sync_copy(rB, acc.at[didx(g + 1)], add=True)
                    return carry

                lax.fori_loop(0, GPT // 2, body, 0)
                plsc.subcore_barrier()
                cnt = 512 if p == 0 else 488
                @pl.when(s < 10)
                def _(jrow=jrow, lo=lo, cnt=cnt):
                    pltpu.sync_copy(
                        acc.at[pl.ds(s * cnt, cnt)],
                        aggh3.at[jrow, pl.ds(lo + s * cnt, cnt)],
                    )
                plsc.subcore_barrier()

    _SC_CACHE["deg"] = _deg_kernel
    _SC_CACHE["agg"] = _agg_kernel
    return _SC_CACHE["deg"], _SC_CACHE["agg"]


def _mm(a, b):
    return lax.dot_general(
        a, b, (((1,), (0,)), ((), ())),
        preferred_element_type=jnp.float32,
        precision=lax.Precision.DEFAULT,
    )


def _tab_halves(x, dpack, w_ref, out_ref, out_dt):
    for i in range(P):
        deg = dpack[:, i] - dpack[:, 6]
        nsrc = lax.rsqrt(jnp.maximum(deg, 1.0))
        hw = _mm(x * nsrc[:, None], w_ref[i]).astype(out_dt)
        out_ref[2 * i] = hw[:, :HF]
        out_ref[2 * i + 1] = hw[:, HF:]


def _a0_body(x_ref, dpack_ref, w_ref, out_ref):
    _tab_halves(x_ref[...], dpack_ref[...], w_ref, out_ref, jnp.float32)


def _combine(emb_ref, wacc_ref, sq_ref):
    wv = jnp.sum(wacc_ref[...] * sq_ref[...], axis=1, keepdims=True) / N
    m = jnp.max(wv)
    ex = jnp.exp(wv - m)
    beta = ex / jnp.sum(ex)
    return (
        emb_ref[0] * beta[0:1]
        + emb_ref[1] * beta[1:2]
        + emb_ref[2] * beta[2:3]
    )


def _a1_body(emb_ref, wacc_ref, sq_ref, dpack_ref, w_ref, out_ref):
    x = _combine(emb_ref, wacc_ref, sq_ref)
    _tab_halves(x, dpack_ref[...], w_ref, out_ref, jnp.float32)


def _c_body(agg_ref, dpack_ref, b_ref, sw_ref, sb_ref, emb_ref, wacc_ref):
    @pl.when(pl.program_id(0) == 0)
    def _():
        wacc_ref[...] = jnp.zeros_like(wacc_ref)

    for i in range(P):
        a = jnp.concatenate(
            [agg_ref[2 * i], agg_ref[2 * i + 1]], axis=1
        ).astype(jnp.float32)
        ndst = lax.rsqrt(jnp.maximum(dpack_ref[:, 3 + i], 1.0))
        e = a * ndst[:, None] + b_ref[i, :][None, :]
        e = jnp.where(e > 0, e, jnp.exp(jnp.minimum(e, 0.0)) - 1.0)
        emb_ref[i] = e
        t = jnp.tanh(_mm(e, sw_ref[...]) + sb_ref[...])
        wacc_ref[i, :] += jnp.sum(t, axis=0)


def _d1_body(emb_ref, wacc_ref, sq_ref, num_ref, w1_ref, b1_ref, w2_ref,
             b2_ref, w3_ref, b3_ref, out_ref):
    x = _combine(emb_ref, wacc_ref, sq_ref) * num_ref[...]
    h1 = jnp.maximum(_mm(x, w1_ref[...]) + b1_ref[...], 0.0)
    h2 = jnp.maximum(_mm(h1, w2_ref[...]) + b2_ref[...], 0.0)
    out_ref[...] = _mm(h2, w3_ref[...]) + b3_ref[...]


_GRID = (N // R,)
_FULL = lambda *shape: pl.BlockSpec(shape, lambda r: tuple(0 for _ in shape))
_DPACK_SPEC = lambda: pl.BlockSpec((R, 8), lambda r: (r, 0))


def _tc_a0(x, dpack, w):
    return pl.pallas_call(
        _a0_body,
        grid=_GRID,
        in_specs=[
            pl.BlockSpec((R, D), lambda r: (r, 0)),
            _DPACK_SPEC(),
            _FULL(P, D, D),
        ],
        out_specs=pl.BlockSpec((6, R, HF), lambda r: (0, r, 0)),
        out_shape=jax.ShapeDtypeStruct((6, N, HF), jnp.float32),
    )(x, dpack, w)


def _tc_a1(emb, wacc, sq, dpack, w):
    return pl.pallas_call(
        _a1_body,
        grid=_GRID,
        in_specs=[
            pl.BlockSpec((P, R, D), lambda r: (0, r, 0)),
            _FULL(P, D),
            _FULL(1, D),
            _DPACK_SPEC(),
            _FULL(P, D, D),
        ],
        out_specs=pl.BlockSpec((6, R, HF), lambda r: (0, r, 0)),
        out_shape=jax.ShapeDtypeStruct((6, N, HF), jnp.float32),
    )(emb, wacc, sq, dpack, w)


def _tc_c(agg, dpack, b, sw, sb):
    return pl.pallas_call(
        _c_body,
        grid=_GRID,
        in_specs=[
            pl.BlockSpec((6, R, HF), lambda r: (0, r, 0)),
            _DPACK_SPEC(),
            _FULL(P, D),
            _FULL(D, D),
            _FULL(1, D),
        ],
        out_specs=[
            pl.BlockSpec((P, R, D), lambda r: (0, r, 0)),
            _FULL(P, D),
        ],
        out_shape=[
            jax.ShapeDtypeStruct((P, N, D), jnp.float32),
            jax.ShapeDtypeStruct((P, D), jnp.float32),
        ],
    )(agg, dpack, b, sw, sb)


def _tc_d1(emb, wacc, sq, numrow, w1, b1, w2, b2, w3, b3):
    return pl.pallas_call(
        _d1_body,
        grid=_GRID,
        in_specs=[
            pl.BlockSpec((P, R, D), lambda r: (0, r, 0)),
            _FULL(P, D),
            _FULL(1, D),
            _FULL(1, D),
            _FULL(D, D),
            _FULL(1, D),
            _FULL(D, 128),
            _FULL(1, 128),
            _FULL(128, 64),
            _FULL(1, 64),
        ],
        out_specs=pl.BlockSpec((R, 64), lambda r: (r, 0)),
        out_shape=jax.ShapeDtypeStruct((N, 64), jnp.float32),
    )(emb, wacc, sq, numrow, w1, b1, w2, b2, w3, b3)


def kernel(h, edge_index_0, edge_index_1, edge_index_2,
           gcn_w_l0, gcn_b_l0, gcn_w_l1, gcn_b_l1,
           sa_w_l0, sa_b_l0, sa_q_l0, sa_w_l1, sa_b_l1, sa_q_l1,
           num, pw1, pb1, pw2, pb2, pw3, pb3):
    pad_src = (jnp.arange(NPADE, dtype=jnp.int32) % 64)
    pad_dst = N + (jnp.arange(NPADE, dtype=jnp.int32) % 64)
    eis = (edge_index_0, edge_index_1, edge_index_2)
    rows = [jnp.concatenate([ei[0], pad_src]) for ei in eis]
    rows += [jnp.concatenate([ei[1], pad_dst]) for ei in eis]
    eir = jnp.stack(rows).reshape(6, EPAD // 128, 128)
    eir = jnp.pad(eir, ((0, 0), (0, EGRP - EPAD // 128), (0, 0)))

    fix = jnp.zeros((N, 1), jnp.float32).at[:64, 0].set(NPADE // 64)

    scale = jnp.arange(D) // (D // 10)
    numrow = num[:, scale]

    sb0 = sa_b_l0.reshape(1, D)
    sb1 = sa_b_l1.reshape(1, D)
    sq0 = sa_q_l0.reshape(1, D)
    sq1 = sa_q_l1.reshape(1, D)
    w1p = jnp.pad(pw1, ((0, 0), (0, 6)))
    b1p = jnp.pad(pb1, (0, 6)).reshape(1, D)
    w2p = jnp.pad(pw2, ((0, 6), (0, 53)))
    b2p = jnp.pad(pb2, (0, 53)).reshape(1, 128)
    w3p = jnp.pad(pw3, ((0, 53), (0, 0)))
    b3p = pb3.reshape(1, 64)

    deg_kernel, agg_kernel = _sc_kernels()
    degs = deg_kernel(eir)[:, :N]
    dpack = jnp.concatenate([degs.T, fix, jnp.zeros((N, 1), jnp.float32)], axis=1)

    tab0 = _tc_a0(h, dpack, gcn_w_l0)
    agg0 = agg_kernel(eir, tab0.reshape(6 * N, HF))
    emb0, wacc0 = _tc_c(agg0, dpack, gcn_b_l0, sa_w_l0, sb0)

    tab1 = _tc_a1(emb0, wacc0, sq0, dpack, gcn_w_l1)
    agg1 = agg_kernel(eir, tab1.reshape(6 * N, HF))
    emb1, wacc1 = _tc_c(agg1, dpack, gcn_b_l1, sa_w_l1, sb1)

    return _tc_d1(emb1, wacc1, sq1, numrow, w1p, b1p, w2p, b2p, w3p, b3p)
